# baseline (speedup 1.0000x reference)
import functools

import jax
import jax.numpy as jnp
from jax.experimental import pallas as pl
from jax.experimental.pallas import tpu as pltpu

_BM = 4096
_NBUF = 3


def _mm_kernel(f_ref, x_ref, o_hbm, o_buf, sems, *, n_steps, n_full, tail):
    i = pl.program_id(0)
    slot = jax.lax.rem(i, _NBUF)

    @pl.when(i >= _NBUF)
    def _wait_slot():
        pltpu.make_async_copy(
            o_buf.at[slot],
            o_hbm.at[pl.ds((i - _NBUF) * _BM, _BM)],
            sems.at[slot]).wait()

    o_buf[slot] = jax.lax.dot_general(
        f_ref[...], x_ref[...],
        dimension_numbers=(((1,), (1,)), ((), ())),
        preferred_element_type=jnp.float32)

    @pl.when(i < n_full)
    def _start_full():
        pltpu.make_async_copy(
            o_buf.at[slot],
            o_hbm.at[pl.ds(i * _BM, _BM)],
            sems.at[slot]).start()

    if tail:
        @pl.when(i == n_full)
        def _start_tail():
            pltpu.make_async_copy(
                o_buf.at[slot, pl.ds(0, tail)],
                o_hbm.at[pl.ds(n_full * _BM, tail)],
                sems.at[slot]).start()

    @pl.when(i == n_steps - 1)
    def _drain():
        for step in range(max(n_steps - _NBUF, 0), n_steps):
            sl = step % _NBUF
            if step < n_full:
                pltpu.make_async_copy(
                    o_buf.at[sl],
                    o_hbm.at[pl.ds(step * _BM, _BM)],
                    sems.at[sl]).wait()
            elif tail:
                pltpu.make_async_copy(
                    o_buf.at[sl, pl.ds(0, tail)],
                    o_hbm.at[pl.ds(n_full * _BM, tail)],
                    sems.at[sl]).wait()


def kernel(inputs, indexes, IoU, update_flag, features):
    B, D = inputs.shape
    M = features.shape[0]
    n_steps = pl.cdiv(M, _BM)
    n_full = M // _BM
    tail = M - n_full * _BM
    ot = pl.pallas_call(
        functools.partial(_mm_kernel, n_steps=n_steps, n_full=n_full,
                          tail=tail),
        grid=(n_steps,),
        in_specs=[
            pl.BlockSpec((_BM, D), lambda i: (i, 0)),
            pl.BlockSpec((B, D), lambda i: (0, 0)),
        ],
        out_specs=pl.BlockSpec(memory_space=pl.ANY),
        out_shape=jax.ShapeDtypeStruct((M, B), jnp.float32),
        scratch_shapes=[
            pltpu.VMEM((_NBUF, _BM, B), jnp.float32),
            pltpu.SemaphoreType.DMA((_NBUF,)),
        ],
    )(features, inputs)
    return ot.T


# speedup vs baseline: 1.0012x; 1.0012x over previous
import jax
import jax.numpy as jnp
from jax.experimental import pallas as pl

_BM = 5120


def _mm_kernel(f_ref, x_ref, o_ref):
    o_ref[...] = jax.lax.dot_general(
        f_ref[...], x_ref[...],
        dimension_numbers=(((1,), (1,)), ((), ())),
        preferred_element_type=jnp.float32)


def kernel(inputs, indexes, IoU, update_flag, features):
    B, D = inputs.shape
    M = features.shape[0]
    ot = pl.pallas_call(
        _mm_kernel,
        grid=(pl.cdiv(M, _BM),),
        in_specs=[
            pl.BlockSpec((_BM, D), lambda i: (i, 0)),
            pl.BlockSpec((B, D), lambda i: (0, 0)),
        ],
        out_specs=pl.BlockSpec((_BM, B), lambda i: (i, 0)),
        out_shape=jax.ShapeDtypeStruct((M, B), jnp.float32),
    )(features, inputs)
    return ot.T
